# baseline (speedup 1.0000x reference)
import functools

import jax
import jax.numpy as jnp
from jax.experimental import pallas as pl
from jax.experimental.pallas import tpu as pltpu

_N = 16384
_C = 1000
_GAMMA = 2.0
_BS = 4096
_CB = 200


def _focal_body(x_ref, t_ref, a_ref, out_ref, m_s, s_s, pe_s):
    i = pl.program_id(0)
    j = pl.program_id(1)
    ns = pl.num_programs(0)
    nc = pl.num_programs(1)
    xb = x_ref[...]
    t = t_ref[0, 0, :]

    mb = jnp.max(xb, axis=0)

    ones = jnp.ones((1, _CB), jnp.float32)

    def sums(m_vec):
        eb = jnp.exp(xb - m_vec[None, :])
        iota = jax.lax.broadcasted_iota(jnp.int32, xb.shape, 0) + j * _CB
        emb = jnp.where(iota == t[None, :], eb, 0.0)
        sb = jax.lax.dot_general(ones, eb, (((1,), (0,)), ((), ())),
                                 preferred_element_type=jnp.float32)
        peb = jax.lax.dot_general(ones, emb, (((1,), (0,)), ((), ())),
                                  preferred_element_type=jnp.float32)
        return sb, peb

    @pl.when(j == 0)
    def _first():
        sb, peb = sums(mb)
        m_s[...] = mb.reshape(1, _BS)
        s_s[...] = sb
        pe_s[...] = peb

    @pl.when(j > 0)
    def _merge():
        m_old = m_s[0, :]
        m_new = jnp.maximum(m_old, mb)
        corr = jnp.exp(m_old - m_new).reshape(1, _BS)
        sb, peb = sums(m_new)
        m_s[...] = m_new.reshape(1, _BS)
        s_s[...] = s_s[...] * corr + sb
        pe_s[...] = pe_s[...] * corr + peb

    @pl.when(j == nc - 1)
    def _finalize():
        a2 = a_ref[...]
        q = jax.lax.shift_right_logical(t, 3)
        r = jax.lax.bitwise_and(t, 7)
        iota_q = jax.lax.broadcasted_iota(jnp.int32, (_C // 8, _BS), 0)
        oh_q = (iota_q == q[None, :]).astype(jnp.float32)
        g = jax.lax.dot_general(a2, oh_q, (((0,), (0,)), ((), ())),
                                preferred_element_type=jnp.float32)
        iota_r = jax.lax.broadcasted_iota(jnp.int32, (8, _BS), 0)
        at = jnp.sum(jnp.where(iota_r == r[None, :], g, 0.0), axis=0)

        p = pe_s[0, :] / s_s[0, :]
        logp = jnp.log(p)
        omp = 1.0 - p
        loss = -at * (omp * omp) * logp
        bsum = jnp.sum(loss, keepdims=True).reshape(1, 1)

        @pl.when(i == 0)
        def _init():
            out_ref[...] = jnp.zeros((1, 1), jnp.float32)

        acc = out_ref[...] + bsum
        out_ref[...] = jnp.where(i == ns - 1, acc * (1.0 / _N), acc)


@jax.jit
def kernel(inputs, targets, alpha):
    ns = _N // _BS
    nc = _C // _CB
    xt_view = inputs.T
    t3 = targets.reshape(ns, 1, _BS)
    a2 = alpha.reshape(_C // 8, 8)
    out = pl.pallas_call(
        _focal_body,
        grid=(ns, nc),
        in_specs=[
            pl.BlockSpec((_CB, _BS), lambda i, j: (j, i)),
            pl.BlockSpec((1, 1, _BS), lambda i, j: (i, 0, 0)),
            pl.BlockSpec((_C // 8, 8), lambda i, j: (0, 0)),
        ],
        out_specs=pl.BlockSpec((1, 1), lambda i, j: (0, 0)),
        out_shape=jax.ShapeDtypeStruct((1, 1), jnp.float32),
        scratch_shapes=[
            pltpu.VMEM((1, _BS), jnp.float32),
            pltpu.VMEM((1, _BS), jnp.float32),
            pltpu.VMEM((1, _BS), jnp.float32),
        ],
    )(xt_view, t3, a2)
    return out[0, 0]


# speedup vs baseline: 1.2772x; 1.2772x over previous
import functools

import jax
import jax.numpy as jnp
from jax.experimental import pallas as pl
from jax.experimental.pallas import tpu as pltpu

_N = 16384
_C = 1000
_GAMMA = 2.0
_B = 4096


def _focal_body(x_ref, t_ref, a_ref, out_ref):
    i = pl.program_id(0)
    nb = pl.num_programs(0)
    x = x_ref[...]
    t = t_ref[0, 0, :]
    a2 = a_ref[...]

    m = jnp.max(x, axis=0)
    e = jnp.exp(x - m[None, :])

    iota = jax.lax.broadcasted_iota(jnp.int32, x.shape, 0)
    onehot = iota == t[None, :]
    em = jnp.where(onehot, e, 0.0)

    ones = jnp.ones((1, _C), jnp.float32)
    s = jax.lax.dot_general(ones, e, (((1,), (0,)), ((), ())),
                            preferred_element_type=jnp.float32)
    pe = jax.lax.dot_general(ones, em, (((1,), (0,)), ((), ())),
                             preferred_element_type=jnp.float32)

    q = jax.lax.shift_right_logical(t, 3)
    r = jax.lax.bitwise_and(t, 7)
    iota_q = jax.lax.broadcasted_iota(jnp.int32, (_C // 8, _B), 0)
    oh_q = (iota_q == q[None, :]).astype(jnp.float32)
    g = jax.lax.dot_general(a2, oh_q, (((0,), (0,)), ((), ())),
                            preferred_element_type=jnp.float32)
    iota_r = jax.lax.broadcasted_iota(jnp.int32, (8, _B), 0)
    at = jnp.sum(jnp.where(iota_r == r[None, :], g, 0.0), axis=0)

    p = pe / s
    logp = jnp.log(p)
    omp = 1.0 - p
    loss = -at[None, :] * (omp * omp) * logp
    bsum = jnp.sum(loss, keepdims=True).reshape(1, 1)

    @pl.when(i == 0)
    def _init():
        out_ref[...] = jnp.zeros((1, 1), jnp.float32)

    acc = out_ref[...] + bsum
    out_ref[...] = jnp.where(i == nb - 1, acc * (1.0 / _N), acc)


@jax.jit
def kernel(inputs, targets, alpha):
    nb = _N // _B
    xt_view = inputs.T
    t3 = targets.reshape(nb, 1, _B)
    a2 = alpha.reshape(_C // 8, 8)
    out = pl.pallas_call(
        _focal_body,
        grid=(nb,),
        in_specs=[
            pl.BlockSpec((_C, _B), lambda i: (0, i)),
            pl.BlockSpec((1, 1, _B), lambda i: (i, 0, 0)),
            pl.BlockSpec((_C // 8, 8), lambda i: (0, 0)),
        ],
        out_specs=pl.BlockSpec((1, 1), lambda i: (0, 0)),
        out_shape=jax.ShapeDtypeStruct((1, 1), jnp.float32),
    )(xt_view, t3, a2)
    return out[0, 0]
